# baseline (speedup 1.0000x reference)
import functools

import jax
import jax.numpy as jnp
from jax import lax
from jax.experimental import pallas as pl
from jax.experimental.pallas import tpu as pltpu
from jax.experimental.pallas import tpu_sc as plsc

_D = 64
_B = 4096
_S = 200
_LN = 128
_SL = 8
_NC, _NS = 2, 16
_NW = _NC * _NS
_NTI = _D // _SL
_NTJ = _B // _LN
_NU = _S * _NTJ
_UPW = _NU // _NW


@functools.cache
def _lookup_call():
    mesh = plsc.VectorSubcoreMesh(core_axis_name="c", subcore_axis_name="s")

    @functools.partial(
        pl.kernel,
        out_type=jax.ShapeDtypeStruct((_S, _NTI, _NTJ, _SL, _LN), jnp.float32),
        mesh=mesh,
        scratch_types=[
            pltpu.VMEM((_UPW, _LN), jnp.int32),
            pltpu.VMEM((_LN, _D), jnp.float32),
            pltpu.VMEM((_LN, _D), jnp.float32),
            pltpu.VMEM((_NTI, _SL, _LN), jnp.float32),
            pltpu.VMEM((_NTI, _SL, _LN), jnp.float32),
            pltpu.SemaphoreType.DMA,
            pltpu.SemaphoreType.DMA,
            pltpu.SemaphoreType.DMA,
            pltpu.SemaphoreType.DMA,
        ],
        compiler_params=pltpu.CompilerParams(
            use_tc_tiling_on_sc=False, needs_layout_passes=False),
    )
    def k(idx_hbm, table_hbm, out_hbm, idx_v, gb0, gb1, tb0, tb1,
          g0, g1, s0, s1):
        wid = lax.axis_index("s") * _NC + lax.axis_index("c")
        u0 = wid * _UPW
        gbuf = (gb0, gb1)
        tbuf = (tb0, tb1)
        gsem = (g0, g1)
        ssem = (s0, s1)
        iota16 = lax.iota(jnp.int32, 16)
        tivecs = [(iota16 + 16 * u) >> 3 for u in range(_D // 16)]
        slvecs = [(iota16 + 16 * u) & (_SL - 1) for u in range(_D // 16)]

        pltpu.sync_copy(idx_hbm.at[pl.ds(u0, _UPW)], idx_v)

        for b in range(2):
            pltpu.async_copy(table_hbm.at[idx_v.at[b]], gbuf[b], gsem[b])

        @pl.loop(0, _UPW // 2)
        def _(p):
            for b in range(2):
                t = p * 2 + b
                u = u0 + t
                s = u // _NTJ
                tj = u - s * _NTJ

                @pl.when(p > 0)
                def _():
                    pltpu.make_async_copy(
                        tbuf[b], out_hbm.at[0, pl.ds(0, _NTI), 0],
                        ssem[b]).wait()

                pltpu.make_async_copy(
                    table_hbm.at[idx_v.at[0]], gbuf[b], gsem[b]).wait()

                @plsc.parallel_loop(0, _LN, unroll=4)
                def _(ln):
                    lnvec = jax.lax.broadcast_in_dim(ln, (16,), ())
                    for u in range(_D // 16):
                        v = gbuf[b][ln, pl.ds(16 * u, 16)]
                        plsc.store_scatter(
                            tbuf[b], [tivecs[u], slvecs[u], lnvec], v)

                @pl.when(t < _UPW - 2)
                def _():
                    pltpu.async_copy(
                        table_hbm.at[idx_v.at[t + 2]], gbuf[b], gsem[b])

                pltpu.async_copy(
                    tbuf[b], out_hbm.at[s, pl.ds(0, _NTI), tj], ssem[b])

        for b in range(2):
            pltpu.make_async_copy(
                tbuf[b], out_hbm.at[0, pl.ds(0, _NTI), 0], ssem[b]).wait()

    return k


def kernel(token_ids, embedding_matrix):
    idxr = token_ids.T.reshape(_NU, _LN) * 2
    table_padded = jnp.pad(embedding_matrix, ((0, 0), (0, _LN - _D)))
    table2 = table_padded.reshape(2 * embedding_matrix.shape[0], _D)
    out5 = _lookup_call()(idxr, table2)
    return out5.transpose(2, 4, 0, 1, 3).reshape(_B, _S, _D)


# speedup vs baseline: 1.7965x; 1.7965x over previous
import functools

import jax
import jax.numpy as jnp
from jax import lax
from jax.experimental import pallas as pl
from jax.experimental.pallas import tpu as pltpu
from jax.experimental.pallas import tpu_sc as plsc

_D = 64
_B = 4096
_S = 200
_LN = 128
_SL = 8
_LNP = _LN + 1
_NC, _NS = 2, 16
_NW = _NC * _NS
_NTI = _D // _SL
_NTJ = _B // _LN
_NU = _S * _NTJ
_UPW = _NU // _NW
_PPW = _UPW // 2


@functools.cache
def _lookup_call():
    mesh = plsc.VectorSubcoreMesh(core_axis_name="c", subcore_axis_name="s")

    @functools.partial(
        pl.kernel,
        out_type=jax.ShapeDtypeStruct((_S, _NTI, _NTJ, _SL, _LN), jnp.float32),
        mesh=mesh,
        scratch_types=[
            pltpu.VMEM((_UPW * _LN,), jnp.int32),
            pltpu.VMEM((2 * _LN, _D), jnp.float32),
            pltpu.VMEM((2 * _LN, _D), jnp.float32),
            pltpu.VMEM((2, _NTI, _SL, _LNP), jnp.float32),
            pltpu.VMEM((2, _NTI, _SL, _LNP), jnp.float32),
            pltpu.SemaphoreType.DMA,
            pltpu.SemaphoreType.DMA,
            pltpu.SemaphoreType.DMA,
            pltpu.SemaphoreType.DMA,
        ],
        compiler_params=pltpu.CompilerParams(
            use_tc_tiling_on_sc=False, needs_layout_passes=False),
    )
    def k(idx_hbm, table_hbm, out_hbm, idx_v, gb0, gb1, tb0, tb1,
          g0, g1, s0, s1):
        wid = lax.axis_index("s") * _NC + lax.axis_index("c")
        u0 = wid * _UPW
        gbuf = (gb0, gb1)
        tbuf = (tb0, tb1)
        gsem = (g0, g1)
        ssem = (s0, s1)
        iota16 = lax.iota(jnp.int32, 16)
        tivecs = [(iota16 + 16 * u) >> 3 for u in range(_D // 16)]
        slvecs = [(iota16 + 16 * u) & (_SL - 1) for u in range(_D // 16)]

        pltpu.sync_copy(idx_hbm.at[pl.ds(u0 * _LN, _UPW * _LN)], idx_v)

        def gather_pair(q, b):
            pltpu.async_copy(
                table_hbm.at[idx_v.at[pl.ds(q * 2 * _LN, 2 * _LN)]],
                gbuf[b], gsem[b])

        def drain_store(b):
            pltpu.make_async_copy(
                tbuf[b].at[0, pl.ds(0, _NTI), pl.ds(0, _SL), pl.ds(0, _LN)],
                out_hbm.at[0, pl.ds(0, _NTI), 0], ssem[b]).wait()

        for b in range(2):
            gather_pair(b, b)

        @pl.loop(0, _PPW // 2)
        def _(p):
            for b in range(2):
                q = p * 2 + b
                u = u0 + 2 * q
                s = u // _NTJ
                tj = u - s * _NTJ

                @pl.when(p > 0)
                def _():
                    drain_store(b)
                    drain_store(b)

                pltpu.make_async_copy(
                    table_hbm.at[idx_v.at[pl.ds(0, 2 * _LN)]],
                    gbuf[b], gsem[b]).wait()

                @plsc.parallel_loop(0, 2 * _LN, unroll=4)
                def _(r):
                    j = r >> 7
                    ln = r & (_LN - 1)
                    jvec = jax.lax.broadcast_in_dim(j, (16,), ())
                    lnvec = jax.lax.broadcast_in_dim(ln, (16,), ())
                    for u in range(_D // 16):
                        v = gbuf[b][r, pl.ds(16 * u, 16)]
                        plsc.store_scatter(
                            tbuf[b], [jvec, tivecs[u], slvecs[u], lnvec], v)

                @pl.when(q < _PPW - 2)
                def _():
                    gather_pair(q + 2, b)

                for j in range(2):
                    pltpu.async_copy(
                        tbuf[b].at[j, pl.ds(0, _NTI), pl.ds(0, _SL),
                                   pl.ds(0, _LN)],
                        out_hbm.at[s, pl.ds(0, _NTI), tj + j], ssem[b])

        for b in range(2):
            drain_store(b)
            drain_store(b)

    return k


def kernel(token_ids, embedding_matrix):
    idx = (token_ids.T.reshape(_NU, _LN) * 2).reshape(_NU * _LN)
    table_padded = jnp.pad(embedding_matrix, ((0, 0), (0, _LN - _D)))
    table2 = table_padded.reshape(2 * embedding_matrix.shape[0], _D)
    out5 = _lookup_call()(idx, table2)
    return out5.transpose(2, 4, 0, 1, 3).reshape(_B, _S, _D)
